# baseline (speedup 1.0000x reference)
import jax
import jax.numpy as jnp
from jax import lax
from jax.experimental import pallas as pl
from jax.experimental.pallas import tpu as pltpu
from jax.experimental.pallas import tpu_sc as plsc

N = 10000
NP = 10240
E = 320000
H = 128
B = 256
NC = 2
NS = 16
NW = NC * NS
RW = NP // NW
EK = 80
NCHT = 256
C0 = 32
C1 = NCHT - C0
RW0 = 160
RW1 = 2 * RW - RW0
EP = NS * NCHT * EK
NBUF = 4
MRING = 8
RZ = NP // NS
RB = 1024
NBLK = NP // RB
VOC = 100000
VB = 4096
VG = (VOC + VB - 1) // VB

_mesh = plsc.VectorSubcoreMesh(core_axis_name="c", subcore_axis_name="s",
                               num_cores=NC, num_subcores=NS)


def _gather_body(idx_hbm, tab_hbm, out_hbm, idx_v, rows_v, sem):
    c = lax.axis_index("c")
    s = lax.axis_index("s")

    def run(base, nrows):
        pltpu.sync_copy(idx_hbm.at[pl.ds(base, nrows)], idx_v.at[pl.ds(0, nrows)])
        pltpu.async_copy(tab_hbm.at[idx_v.at[pl.ds(0, nrows)]],
                         rows_v.at[pl.ds(0, nrows)], sem).wait()
        pltpu.sync_copy(rows_v.at[pl.ds(0, nrows)], out_hbm.at[pl.ds(base, nrows)])

    pl.when(c == 0)(lambda: run(s * RW0, RW0))
    pl.when(c == 1)(lambda: run(NS * RW0 + s * RW1, RW1))


_embed_gather = pl.kernel(
    _gather_body,
    out_type=jax.ShapeDtypeStruct((NP, H), jnp.float32),
    mesh=_mesh,
    scratch_types=[
        pltpu.VMEM((RW1,), jnp.int32),
        pltpu.VMEM((RW1, H), jnp.float32),
        pltpu.SemaphoreType.DMA,
    ],
)


def _scale_chunk(rows_ref, ew_ref):
    def grp(g, carry):
        w16 = ew_ref[pl.ds(g * 16, 16)]
        for l in range(16):
            w = w16[l]
            r = g * 16 + l
            for cc in range(H // 16):
                sl = pl.ds(cc * 16, 16)
                rows_ref[r, sl] = rows_ref[r, sl] * w
        return carry

    lax.fori_loop(0, EK // 16, grp, 0)


def _edge_body(h_hbm, src_hbm, dst_hbm, ew_hbm, out_hbm,
               src_v, dst_v, ew_v, rows_v, agg_sh, *sems):
    gsems = sems[:NBUF]
    ssems = sems[NBUF:NBUF + MRING]
    msems = sems[NBUF + MRING:]
    c = lax.axis_index("c")
    s = lax.axis_index("s")

    def meta_issue(slot, t, wch):
        base = (wch + t) * EK
        pltpu.async_copy(src_hbm.at[pl.ds(base, EK)], src_v.at[slot], msems[slot])
        pltpu.async_copy(dst_hbm.at[pl.ds(base, EK)], dst_v.at[slot], msems[slot])
        pltpu.async_copy(ew_hbm.at[pl.ds(base, EK)], ew_v.at[slot], msems[slot])

    def meta_wait(slot, t, wch):
        base = (wch + t) * EK
        pltpu.make_async_copy(src_hbm.at[pl.ds(base, EK)], src_v.at[slot],
                              msems[slot]).wait()
        pltpu.make_async_copy(dst_hbm.at[pl.ds(base, EK)], dst_v.at[slot],
                              msems[slot]).wait()
        pltpu.make_async_copy(ew_hbm.at[pl.ds(base, EK)], ew_v.at[slot],
                              msems[slot]).wait()

    def gather_issue(mslot, rslot):
        pltpu.async_copy(h_hbm.at[src_v.at[mslot]], rows_v.at[rslot],
                         gsems[rslot])

    def gather_wait(mslot, rslot):
        pltpu.make_async_copy(h_hbm.at[src_v.at[mslot]], rows_v.at[rslot],
                              gsems[rslot]).wait()

    def scatter_issue(mslot, rslot):
        pltpu.async_copy(rows_v.at[rslot], agg_sh.at[dst_v.at[mslot]],
                         ssems[mslot], add=True)

    def scatter_wait(mslot, rslot):
        pltpu.make_async_copy(rows_v.at[rslot], agg_sh.at[dst_v.at[mslot]],
                              ssems[mslot]).wait()

    zb = rows_v.at[0]

    def zrow(i, carry):
        for cc in range(H // 16):
            zb[i, pl.ds(cc * 16, 16)] = jnp.zeros((16,), jnp.float32)
        return carry

    lax.fori_loop(0, EK, zrow, 0)

    def zcopy(i, carry):
        pltpu.sync_copy(zb, agg_sh.at[pl.ds(s * RZ + i * EK, EK)])
        return carry

    lax.fori_loop(0, RZ // EK, zcopy, 0)
    plsc.subcore_barrier()

    wch = jnp.where(c == 0, s * C0, NS * C0 + s * C1)
    nch = jnp.where(c == 0, C0, C1)
    kmax = nch // MRING

    for slot in range(5):
        meta_issue(slot, slot, wch)
    meta_wait(0, 0, wch)
    gather_issue(0, 0)

    def body(k, carry):
        for b in range(MRING):
            t = k * MRING + b
            rb = b % NBUF
            nrb = (b + 1) % NBUF
            nmb = (b + 1) % MRING
            wmb = (b + 5) % MRING
            wrb = (b + 1) % NBUF

            def w_scat(wmb=wmb, wrb=wrb):
                scatter_wait(wmb, wrb)

            def pf_gather(nmb=nmb, nrb=nrb, t=t):
                meta_wait(nmb, t + 1, wch)
                gather_issue(nmb, nrb)

            def pf_meta(wmb=wmb, t=t):
                meta_issue(wmb, t + 5, wch)

            if b < 3:
                pl.when(k > 0)(w_scat)
            else:
                w_scat()
            if b < MRING - 1:
                pf_gather()
            else:
                pl.when(k < kmax - 1)(pf_gather)
            if b < 3:
                pf_meta()
            else:
                pl.when(k < kmax - 1)(pf_meta)
            gather_wait(b, rb)
            _scale_chunk(rows_v.at[rb], ew_v.at[b])
            scatter_issue(b, rb)
        return carry

    lax.fori_loop(0, kmax, body, 0)
    for toff, rslot, mslot in ((3, 1, 5), (2, 2, 6), (1, 3, 7)):
        del toff
        scatter_wait(mslot, rslot)
    plsc.subcore_barrier()

    def wb(i, carry):
        pltpu.sync_copy(agg_sh.at[pl.ds(s * RZ + i * EK, EK)],
                        out_hbm.at[c, pl.ds(s * RZ + i * EK, EK)])
        return carry

    lax.fori_loop(0, RZ // EK, wb, 0)


_edge_pass = pl.kernel(
    _edge_body,
    out_type=jax.ShapeDtypeStruct((NC, NP, H), jnp.float32),
    mesh=_mesh,
    scratch_types=[
        pltpu.VMEM((MRING, EK), jnp.int32),
        pltpu.VMEM((MRING, EK), jnp.int32),
        pltpu.VMEM((MRING, EK), jnp.float32),
        pltpu.VMEM((NBUF, EK, H), jnp.float32),
        pltpu.VMEM_SHARED((NP, H), jnp.float32),
    ] + [pltpu.SemaphoreType.DMA] * (NBUF + 2 * MRING),
)


def _gru_body(a_ref, h_ref, bat_ref, nxt_ref, wggc_ref, wih_ref, whh_ref,
              bih_ref, bhh_ref, v_ref, sl_ref, cnt_ref):
    i = pl.program_id(0)
    agg = a_ref[0] + a_ref[1]
    h = h_ref[...]
    wc = lax.dot_general(wggc_ref[...], wih_ref[...], (((1,), (1,)), ((), ())))
    gi = lax.dot(agg, wc, preferred_element_type=jnp.float32) + bih_ref[...]
    gh = lax.dot_general(h, whh_ref[...], (((1,), (1,)), ((), ()))) + bhh_ref[...]
    r = jax.nn.sigmoid(gi[:, :H] + gh[:, :H])
    zg = jax.nn.sigmoid(gi[:, H:2 * H] + gh[:, H:2 * H])
    ng = jnp.tanh(gi[:, 2 * H:] + r * gh[:, 2 * H:])
    v = (1.0 - zg) * ng + zg * h
    v_ref[...] = v
    bat = bat_ref[0, 0, :]
    oh = (bat[:, None] == lax.broadcasted_iota(jnp.int32, (RB, B), 1)).astype(jnp.float32)
    islast = (bat != nxt_ref[0, 0, :]).astype(jnp.float32)

    @pl.when(i == 0)
    def _():
        sl_ref[...] = jnp.zeros_like(sl_ref)
        cnt_ref[...] = jnp.zeros_like(cnt_ref)

    sl_ref[...] += lax.dot_general(oh * islast[:, None], v, (((0,), (0,)), ((), ())))
    cnt_ref[...] += lax.dot_general(oh, jnp.ones((RB, 1), jnp.float32),
                                    (((0,), (0,)), ((), ())))


def _gru(agg2, h, bat2, nxt2, wggc, wih, whh, bih, bhh):
    return pl.pallas_call(
        _gru_body,
        grid=(NBLK,),
        in_specs=[
            pl.BlockSpec((NC, RB, H), lambda i: (0, i, 0)),
            pl.BlockSpec((RB, H), lambda i: (i, 0)),
            pl.BlockSpec((1, 1, RB), lambda i: (i, 0, 0)),
            pl.BlockSpec((1, 1, RB), lambda i: (i, 0, 0)),
            pl.BlockSpec((H, H), lambda i: (0, 0)),
            pl.BlockSpec((3 * H, H), lambda i: (0, 0)),
            pl.BlockSpec((3 * H, H), lambda i: (0, 0)),
            pl.BlockSpec((1, 3 * H), lambda i: (0, 0)),
            pl.BlockSpec((1, 3 * H), lambda i: (0, 0)),
        ],
        out_specs=[
            pl.BlockSpec((RB, H), lambda i: (i, 0)),
            pl.BlockSpec((B, H), lambda i: (0, 0)),
            pl.BlockSpec((B, 1), lambda i: (0, 0)),
        ],
        out_shape=[
            jax.ShapeDtypeStruct((NP, H), jnp.float32),
            jax.ShapeDtypeStruct((B, H), jnp.float32),
            jax.ShapeDtypeStruct((B, 1), jnp.float32),
        ],
    )(agg2, h, bat2, nxt2, wggc, wih, whh, bih, bhh)


def _att_body(v_ref, bat_ref, sl_ref, cnt_ref, vh_ref, w1_ref, b1_ref,
              w2_ref, b2_ref, qw_ref, qb_ref, w3_ref, b3_ref,
              sg_ref, sh_ref):
    i = pl.program_id(0)
    sl = sl_ref[...] + (cnt_ref[...] == 0.0).astype(jnp.float32) * vh_ref[0:1, :]
    bat = bat_ref[0, 0, :]
    oh = (bat[:, None] == lax.broadcasted_iota(jnp.int32, (RB, B), 1)).astype(jnp.float32)
    v = v_ref[...]
    vn = lax.dot(oh, sl, preferred_element_type=jnp.float32)
    pre = (lax.dot_general(vn, w1_ref[...], (((1,), (1,)), ((), ()))) + b1_ref[...]
           + lax.dot_general(v, w2_ref[...], (((1,), (1,)), ((), ()))) + b2_ref[...])
    sig = jax.nn.sigmoid(pre)
    alpha = jnp.sum(sig * qw_ref[...], axis=1, keepdims=True) + qb_ref[...]

    @pl.when(i == 0)
    def _():
        sg_ref[...] = jnp.zeros_like(sg_ref)

    sg_ref[...] += lax.dot_general(oh, alpha * v, (((0,), (0,)), ((), ())))

    @pl.when(i == NBLK - 1)
    def _():
        cat = jnp.concatenate([sl, sg_ref[...]], axis=1)
        sh_ref[...] = (lax.dot_general(cat, w3_ref[...], (((1,), (1,)), ((), ())))
                       + b3_ref[...])


def _att(v, bat2, sl, cnt, w1, b1, w2, b2, qw, qb, w3, b3):
    return pl.pallas_call(
        _att_body,
        grid=(NBLK,),
        in_specs=[
            pl.BlockSpec((RB, H), lambda i: (i, 0)),
            pl.BlockSpec((1, 1, RB), lambda i: (i, 0, 0)),
            pl.BlockSpec((B, H), lambda i: (0, 0)),
            pl.BlockSpec((B, 1), lambda i: (0, 0)),
            pl.BlockSpec((8, H), lambda i: (0, 0)),
            pl.BlockSpec((H, H), lambda i: (0, 0)),
            pl.BlockSpec((1, H), lambda i: (0, 0)),
            pl.BlockSpec((H, H), lambda i: (0, 0)),
            pl.BlockSpec((1, H), lambda i: (0, 0)),
            pl.BlockSpec((1, H), lambda i: (0, 0)),
            pl.BlockSpec((1, 1), lambda i: (0, 0)),
            pl.BlockSpec((H, 2 * H), lambda i: (0, 0)),
            pl.BlockSpec((1, H), lambda i: (0, 0)),
        ],
        out_specs=[
            pl.BlockSpec((B, H), lambda i: (0, 0)),
            pl.BlockSpec((B, H), lambda i: (0, 0)),
        ],
        out_shape=[
            jax.ShapeDtypeStruct((B, H), jnp.float32),
            jax.ShapeDtypeStruct((B, H), jnp.float32),
        ],
    )(v, bat2, sl, cnt, v, w1, b1, w2, b2, qw, qb, w3, b3)


def _z_body(sh_ref, emb_ref, z_ref):
    z_ref[...] = lax.dot_general(sh_ref[...], emb_ref[...], (((1,), (1,)), ((), ())))


def _zproj(sh, embed):
    return pl.pallas_call(
        _z_body,
        grid=(VG,),
        in_specs=[
            pl.BlockSpec((B, H), lambda i: (0, 0)),
            pl.BlockSpec((VB, H), lambda i: (i, 0)),
        ],
        out_specs=pl.BlockSpec((B, VB), lambda i: (0, i)),
        out_shape=jax.ShapeDtypeStruct((B, VOC), jnp.float32),
    )(sh, embed)


def kernel(x, edge_index, edge_weight, batch, embed, W_ggc, W_ih, W_hh, b_ih, b_hh,
           W1_w, W1_b, W2_w, W2_b, q_w, q_b, W3_w, W3_b):
    x_pad = jnp.concatenate([x.astype(jnp.int32), jnp.zeros((NP - N,), jnp.int32)])
    h = _embed_gather(x_pad, embed)
    epad = jnp.arange(EP - E, dtype=jnp.int32) % N
    src = jnp.concatenate([edge_index[0].astype(jnp.int32), epad])
    dst = jnp.concatenate([edge_index[1].astype(jnp.int32), epad])
    ew2 = jnp.concatenate([edge_weight, jnp.zeros((EP - E,), jnp.float32)])
    agg2 = _edge_pass(h, src, dst, ew2)

    batch_pad = jnp.concatenate([batch.astype(jnp.int32),
                                 jnp.full((NP - N,), B, jnp.int32)])
    batch_nxt = jnp.concatenate([batch_pad[1:], jnp.full((1,), B, jnp.int32)])
    bat2 = batch_pad.reshape(NBLK, 1, RB)
    nxt2 = batch_nxt.reshape(NBLK, 1, RB)
    v, sl_raw, cnt = _gru(agg2, h, bat2, nxt2, W_ggc, W_ih, W_hh,
                          b_ih.reshape(1, 3 * H), b_hh.reshape(1, 3 * H))
    sg, sh = _att(v, bat2, sl_raw, cnt, W1_w, W1_b.reshape(1, H),
                  W2_w, W2_b.reshape(1, H), q_w.reshape(1, H),
                  q_b.reshape(1, 1), W3_w, W3_b.reshape(1, H))
    del sg
    return _zproj(sh, embed)


# speedup vs baseline: 1.2774x; 1.2774x over previous
import jax
import jax.numpy as jnp
from jax import lax
from jax.experimental import pallas as pl
from jax.experimental.pallas import tpu as pltpu
from jax.experimental.pallas import tpu_sc as plsc

N = 10000
NP = 10240
E = 320000
H = 128
B = 256
NC = 2
NS = 16
NW = NC * NS
RW = NP // NW
EK = 80
NCHT = 256
C0 = 128
C1 = NCHT - C0
EP = NS * NCHT * EK
NBUF = 4
MRING = 8
RZ = NP // NS
RB = 1024
NBLK = NP // RB
VOC = 100000
VB = 4096
VG = (VOC + VB - 1) // VB

_mesh = plsc.VectorSubcoreMesh(core_axis_name="c", subcore_axis_name="s",
                               num_cores=NC, num_subcores=NS)


def _gather_body(idx_hbm, tab_hbm, out_hbm, idx_v, rows_v, sem):
    wid = lax.axis_index("s") * NC + lax.axis_index("c")
    base = wid * RW
    pltpu.sync_copy(idx_hbm.at[pl.ds(base, RW)], idx_v)
    pltpu.async_copy(tab_hbm.at[idx_v], rows_v, sem).wait()
    pltpu.sync_copy(rows_v, out_hbm.at[pl.ds(base, RW)])


_embed_gather = pl.kernel(
    _gather_body,
    out_type=jax.ShapeDtypeStruct((NP, H), jnp.float32),
    mesh=_mesh,
    scratch_types=[
        pltpu.VMEM((RW,), jnp.int32),
        pltpu.VMEM((RW, H), jnp.float32),
        pltpu.SemaphoreType.DMA,
    ],
)


def _scale_chunk(rows_ref, ew_ref):
    def grp(g, carry):
        w16 = ew_ref[pl.ds(g * 16, 16)]
        for l in range(16):
            w = w16[l]
            r = g * 16 + l
            for cc in range(H // 16):
                sl = pl.ds(cc * 16, 16)
                rows_ref[r, sl] = rows_ref[r, sl] * w
        return carry

    lax.fori_loop(0, EK // 16, grp, 0)


def _edge_body(h_hbm, src_hbm, dst_hbm, ew_hbm, out_hbm,
               src_v, dst_v, ew_v, rows_v, agg_sh, *sems):
    gsems = sems[:NBUF]
    ssems = sems[NBUF:NBUF + MRING]
    msems = sems[NBUF + MRING:]
    c = lax.axis_index("c")
    s = lax.axis_index("s")

    def meta_issue(slot, t, wch):
        base = (wch + t) * EK
        pltpu.async_copy(src_hbm.at[pl.ds(base, EK)], src_v.at[slot], msems[slot])
        pltpu.async_copy(dst_hbm.at[pl.ds(base, EK)], dst_v.at[slot], msems[slot])
        pltpu.async_copy(ew_hbm.at[pl.ds(base, EK)], ew_v.at[slot], msems[slot])

    def meta_wait(slot, t, wch):
        base = (wch + t) * EK
        pltpu.make_async_copy(src_hbm.at[pl.ds(base, EK)], src_v.at[slot],
                              msems[slot]).wait()
        pltpu.make_async_copy(dst_hbm.at[pl.ds(base, EK)], dst_v.at[slot],
                              msems[slot]).wait()
        pltpu.make_async_copy(ew_hbm.at[pl.ds(base, EK)], ew_v.at[slot],
                              msems[slot]).wait()

    def gather_issue(mslot, rslot):
        pltpu.async_copy(h_hbm.at[src_v.at[mslot]], rows_v.at[rslot],
                         gsems[rslot])

    def gather_wait(mslot, rslot):
        pltpu.make_async_copy(h_hbm.at[src_v.at[mslot]], rows_v.at[rslot],
                              gsems[rslot]).wait()

    def scatter_issue(mslot, rslot):
        pltpu.async_copy(rows_v.at[rslot], agg_sh.at[dst_v.at[mslot]],
                         ssems[mslot], add=True)

    def scatter_wait(mslot, rslot):
        pltpu.make_async_copy(rows_v.at[rslot], agg_sh.at[dst_v.at[mslot]],
                              ssems[mslot]).wait()

    zb = rows_v.at[0]

    def zrow(i, carry):
        for cc in range(H // 16):
            zb[i, pl.ds(cc * 16, 16)] = jnp.zeros((16,), jnp.float32)
        return carry

    lax.fori_loop(0, EK, zrow, 0)

    def zcopy(i, carry):
        pltpu.sync_copy(zb, agg_sh.at[pl.ds(s * RZ + i * EK, EK)])
        return carry

    lax.fori_loop(0, RZ // EK, zcopy, 0)
    plsc.subcore_barrier()

    wch = jnp.where(c == 0, s * C0, NS * C0 + s * C1)
    nch = jnp.where(c == 0, C0, C1)
    kmax = nch // MRING

    for slot in range(5):
        meta_issue(slot, slot, wch)
    meta_wait(0, 0, wch)
    gather_issue(0, 0)

    def body(k, carry):
        for b in range(MRING):
            t = k * MRING + b
            rb = b % NBUF
            nrb = (b + 1) % NBUF
            nmb = (b + 1) % MRING
            wmb = (b + 5) % MRING
            wrb = (b + 1) % NBUF

            def w_scat(wmb=wmb, wrb=wrb):
                scatter_wait(wmb, wrb)

            def pf_gather(nmb=nmb, nrb=nrb, t=t):
                meta_wait(nmb, t + 1, wch)
                gather_issue(nmb, nrb)

            def pf_meta(wmb=wmb, t=t):
                meta_issue(wmb, t + 5, wch)

            if b < 3:
                pl.when(k > 0)(w_scat)
            else:
                w_scat()
            if b < MRING - 1:
                pf_gather()
            else:
                pl.when(k < kmax - 1)(pf_gather)
            if b < 3:
                pf_meta()
            else:
                pl.when(k < kmax - 1)(pf_meta)
            gather_wait(b, rb)
            _scale_chunk(rows_v.at[rb], ew_v.at[b])
            scatter_issue(b, rb)
        return carry

    lax.fori_loop(0, kmax, body, 0)
    for toff, rslot, mslot in ((3, 1, 5), (2, 2, 6), (1, 3, 7)):
        del toff
        scatter_wait(mslot, rslot)
    plsc.subcore_barrier()

    def wb(i, carry):
        pltpu.sync_copy(agg_sh.at[pl.ds(s * RZ + i * EK, EK)],
                        out_hbm.at[c, pl.ds(s * RZ + i * EK, EK)])
        return carry

    lax.fori_loop(0, RZ // EK, wb, 0)


_edge_pass = pl.kernel(
    _edge_body,
    out_type=jax.ShapeDtypeStruct((NC, NP, H), jnp.float32),
    mesh=_mesh,
    scratch_types=[
        pltpu.VMEM((MRING, EK), jnp.int32),
        pltpu.VMEM((MRING, EK), jnp.int32),
        pltpu.VMEM((MRING, EK), jnp.float32),
        pltpu.VMEM((NBUF, EK, H), jnp.float32),
        pltpu.VMEM_SHARED((NP, H), jnp.float32),
    ] + [pltpu.SemaphoreType.DMA] * (NBUF + 2 * MRING),
)


def _gru_body(a_ref, h_ref, bat_ref, nxt_ref, wggc_ref, wih_ref, whh_ref,
              bih_ref, bhh_ref, v_ref, sl_ref, cnt_ref):
    i = pl.program_id(0)
    agg = a_ref[0] + a_ref[1]
    h = h_ref[...]
    wc = lax.dot_general(wggc_ref[...], wih_ref[...], (((1,), (1,)), ((), ())))
    gi = lax.dot(agg, wc, preferred_element_type=jnp.float32) + bih_ref[...]
    gh = lax.dot_general(h, whh_ref[...], (((1,), (1,)), ((), ()))) + bhh_ref[...]
    r = jax.nn.sigmoid(gi[:, :H] + gh[:, :H])
    zg = jax.nn.sigmoid(gi[:, H:2 * H] + gh[:, H:2 * H])
    ng = jnp.tanh(gi[:, 2 * H:] + r * gh[:, 2 * H:])
    v = (1.0 - zg) * ng + zg * h
    v_ref[...] = v
    bat = bat_ref[0, 0, :]
    oh = (bat[:, None] == lax.broadcasted_iota(jnp.int32, (RB, B), 1)).astype(jnp.float32)
    islast = (bat != nxt_ref[0, 0, :]).astype(jnp.float32)

    @pl.when(i == 0)
    def _():
        sl_ref[...] = jnp.zeros_like(sl_ref)
        cnt_ref[...] = jnp.zeros_like(cnt_ref)

    sl_ref[...] += lax.dot_general(oh * islast[:, None], v, (((0,), (0,)), ((), ())))
    cnt_ref[...] += lax.dot_general(oh, jnp.ones((RB, 1), jnp.float32),
                                    (((0,), (0,)), ((), ())))


def _gru(agg2, h, bat2, nxt2, wggc, wih, whh, bih, bhh):
    return pl.pallas_call(
        _gru_body,
        grid=(NBLK,),
        in_specs=[
            pl.BlockSpec((NC, RB, H), lambda i: (0, i, 0)),
            pl.BlockSpec((RB, H), lambda i: (i, 0)),
            pl.BlockSpec((1, 1, RB), lambda i: (i, 0, 0)),
            pl.BlockSpec((1, 1, RB), lambda i: (i, 0, 0)),
            pl.BlockSpec((H, H), lambda i: (0, 0)),
            pl.BlockSpec((3 * H, H), lambda i: (0, 0)),
            pl.BlockSpec((3 * H, H), lambda i: (0, 0)),
            pl.BlockSpec((1, 3 * H), lambda i: (0, 0)),
            pl.BlockSpec((1, 3 * H), lambda i: (0, 0)),
        ],
        out_specs=[
            pl.BlockSpec((RB, H), lambda i: (i, 0)),
            pl.BlockSpec((B, H), lambda i: (0, 0)),
            pl.BlockSpec((B, 1), lambda i: (0, 0)),
        ],
        out_shape=[
            jax.ShapeDtypeStruct((NP, H), jnp.float32),
            jax.ShapeDtypeStruct((B, H), jnp.float32),
            jax.ShapeDtypeStruct((B, 1), jnp.float32),
        ],
    )(agg2, h, bat2, nxt2, wggc, wih, whh, bih, bhh)


def _att_body(v_ref, bat_ref, sl_ref, cnt_ref, vh_ref, w1_ref, b1_ref,
              w2_ref, b2_ref, qw_ref, qb_ref, w3_ref, b3_ref,
              sg_ref, sh_ref):
    i = pl.program_id(0)
    sl = sl_ref[...] + (cnt_ref[...] == 0.0).astype(jnp.float32) * vh_ref[0:1, :]
    bat = bat_ref[0, 0, :]
    oh = (bat[:, None] == lax.broadcasted_iota(jnp.int32, (RB, B), 1)).astype(jnp.float32)
    v = v_ref[...]
    vn = lax.dot(oh, sl, preferred_element_type=jnp.float32)
    pre = (lax.dot_general(vn, w1_ref[...], (((1,), (1,)), ((), ()))) + b1_ref[...]
           + lax.dot_general(v, w2_ref[...], (((1,), (1,)), ((), ()))) + b2_ref[...])
    sig = jax.nn.sigmoid(pre)
    alpha = jnp.sum(sig * qw_ref[...], axis=1, keepdims=True) + qb_ref[...]

    @pl.when(i == 0)
    def _():
        sg_ref[...] = jnp.zeros_like(sg_ref)

    sg_ref[...] += lax.dot_general(oh, alpha * v, (((0,), (0,)), ((), ())))

    @pl.when(i == NBLK - 1)
    def _():
        cat = jnp.concatenate([sl, sg_ref[...]], axis=1)
        sh_ref[...] = (lax.dot_general(cat, w3_ref[...], (((1,), (1,)), ((), ())))
                       + b3_ref[...])


def _att(v, bat2, sl, cnt, w1, b1, w2, b2, qw, qb, w3, b3):
    return pl.pallas_call(
        _att_body,
        grid=(NBLK,),
        in_specs=[
            pl.BlockSpec((RB, H), lambda i: (i, 0)),
            pl.BlockSpec((1, 1, RB), lambda i: (i, 0, 0)),
            pl.BlockSpec((B, H), lambda i: (0, 0)),
            pl.BlockSpec((B, 1), lambda i: (0, 0)),
            pl.BlockSpec((8, H), lambda i: (0, 0)),
            pl.BlockSpec((H, H), lambda i: (0, 0)),
            pl.BlockSpec((1, H), lambda i: (0, 0)),
            pl.BlockSpec((H, H), lambda i: (0, 0)),
            pl.BlockSpec((1, H), lambda i: (0, 0)),
            pl.BlockSpec((1, H), lambda i: (0, 0)),
            pl.BlockSpec((1, 1), lambda i: (0, 0)),
            pl.BlockSpec((H, 2 * H), lambda i: (0, 0)),
            pl.BlockSpec((1, H), lambda i: (0, 0)),
        ],
        out_specs=[
            pl.BlockSpec((B, H), lambda i: (0, 0)),
            pl.BlockSpec((B, H), lambda i: (0, 0)),
        ],
        out_shape=[
            jax.ShapeDtypeStruct((B, H), jnp.float32),
            jax.ShapeDtypeStruct((B, H), jnp.float32),
        ],
    )(v, bat2, sl, cnt, v, w1, b1, w2, b2, qw, qb, w3, b3)


def _z_body(sh_ref, emb_ref, z_ref):
    z_ref[...] = lax.dot_general(sh_ref[...], emb_ref[...], (((1,), (1,)), ((), ())))


def _zproj(sh, embed):
    return pl.pallas_call(
        _z_body,
        grid=(VG,),
        in_specs=[
            pl.BlockSpec((B, H), lambda i: (0, 0)),
            pl.BlockSpec((VB, H), lambda i: (i, 0)),
        ],
        out_specs=pl.BlockSpec((B, VB), lambda i: (0, i)),
        out_shape=jax.ShapeDtypeStruct((B, VOC), jnp.float32),
    )(sh, embed)


def kernel(x, edge_index, edge_weight, batch, embed, W_ggc, W_ih, W_hh, b_ih, b_hh,
           W1_w, W1_b, W2_w, W2_b, q_w, q_b, W3_w, W3_b):
    x_pad = jnp.concatenate([x.astype(jnp.int32),
                             jnp.arange(NP - N, dtype=jnp.int32)])
    h = _embed_gather(x_pad, embed)
    epad = jnp.arange(EP - E, dtype=jnp.int32) % N
    src = jnp.concatenate([edge_index[0].astype(jnp.int32), epad])
    dst = jnp.concatenate([edge_index[1].astype(jnp.int32), epad])
    ew2 = jnp.concatenate([edge_weight, jnp.zeros((EP - E,), jnp.float32)])
    agg2 = _edge_pass(h, src, dst, ew2)

    batch_pad = jnp.concatenate([batch.astype(jnp.int32),
                                 jnp.full((NP - N,), B, jnp.int32)])
    batch_nxt = jnp.concatenate([batch_pad[1:], jnp.full((1,), B, jnp.int32)])
    bat2 = batch_pad.reshape(NBLK, 1, RB)
    nxt2 = batch_nxt.reshape(NBLK, 1, RB)
    v, sl_raw, cnt = _gru(agg2, h, bat2, nxt2, W_ggc, W_ih, W_hh,
                          b_ih.reshape(1, 3 * H), b_hh.reshape(1, 3 * H))
    sg, sh = _att(v, bat2, sl_raw, cnt, W1_w, W1_b.reshape(1, H),
                  W2_w, W2_b.reshape(1, H), q_w.reshape(1, H),
                  q_b.reshape(1, 1), W3_w, W3_b.reshape(1, H))
    del sg
    return _zproj(sh, embed)


# speedup vs baseline: 1.3283x; 1.0398x over previous
import jax
import jax.numpy as jnp
from jax import lax
from jax.experimental import pallas as pl
from jax.experimental.pallas import tpu as pltpu
from jax.experimental.pallas import tpu_sc as plsc

N = 10000
NP = 10240
E = 320000
H = 128
B = 256
NC = 2
NS = 16
NW = NC * NS
RW = NP // NW
EK = 80
NCHT = 256
C0 = 128
C1 = NCHT - C0
EP = NS * NCHT * EK
NBUF = 4
MRING = 8
RZ = NP // NS
RB = 1024
NBLK = NP // RB
VOC = 100000
VB = 4096
VG = (VOC + VB - 1) // VB

_mesh = plsc.VectorSubcoreMesh(core_axis_name="c", subcore_axis_name="s",
                               num_cores=NC, num_subcores=NS)


def _gather_body(idx_hbm, tab_hbm, out_hbm, idx_v, rows_v, sem):
    wid = lax.axis_index("s") * NC + lax.axis_index("c")
    base = wid * RW
    pltpu.sync_copy(idx_hbm.at[pl.ds(base, RW)], idx_v)
    pltpu.async_copy(tab_hbm.at[idx_v], rows_v, sem).wait()
    pltpu.sync_copy(rows_v, out_hbm.at[pl.ds(base, RW)])


_embed_gather = pl.kernel(
    _gather_body,
    out_type=jax.ShapeDtypeStruct((NP, H), jnp.float32),
    mesh=_mesh,
    scratch_types=[
        pltpu.VMEM((RW,), jnp.int32),
        pltpu.VMEM((RW, H), jnp.float32),
        pltpu.SemaphoreType.DMA,
    ],
)


def _scale_chunk(rows_ref, ew_ref):
    def grp(g, carry):
        w16 = ew_ref[pl.ds(g * 16, 16)]
        for l in range(16):
            w = w16[l]
            r = g * 16 + l
            for cc in range(H // 16):
                sl = pl.ds(cc * 16, 16)
                rows_ref[r, sl] = rows_ref[r, sl] * w
        return carry

    lax.fori_loop(0, EK // 16, grp, 0)


def _edge_body(h_hbm, src_hbm, dst_hbm, ew_hbm, out_hbm,
               src_v, dst_v, ew_v, rows_v, agg_sh, *sems):
    gsems = sems[:NBUF]
    ssems = sems[NBUF:NBUF + MRING]
    msems = sems[NBUF + MRING:]
    c = lax.axis_index("c")
    s = lax.axis_index("s")

    def meta_issue(slot, t, wch):
        base = (wch + t) * EK
        pltpu.async_copy(src_hbm.at[pl.ds(base, EK)], src_v.at[slot], msems[slot])
        pltpu.async_copy(dst_hbm.at[pl.ds(base, EK)], dst_v.at[slot], msems[slot])
        pltpu.async_copy(ew_hbm.at[pl.ds(base, EK)], ew_v.at[slot], msems[slot])

    def meta_wait(slot, t, wch):
        base = (wch + t) * EK
        pltpu.make_async_copy(src_hbm.at[pl.ds(base, EK)], src_v.at[slot],
                              msems[slot]).wait()
        pltpu.make_async_copy(dst_hbm.at[pl.ds(base, EK)], dst_v.at[slot],
                              msems[slot]).wait()
        pltpu.make_async_copy(ew_hbm.at[pl.ds(base, EK)], ew_v.at[slot],
                              msems[slot]).wait()

    def gather_issue(mslot, rslot):
        pltpu.async_copy(h_hbm.at[src_v.at[mslot]], rows_v.at[rslot],
                         gsems[rslot])

    def gather_wait(mslot, rslot):
        pltpu.make_async_copy(h_hbm.at[src_v.at[mslot]], rows_v.at[rslot],
                              gsems[rslot]).wait()

    def scatter_issue(mslot, rslot):
        pltpu.async_copy(rows_v.at[rslot], agg_sh.at[dst_v.at[mslot]],
                         ssems[mslot], add=True)

    def scatter_wait(mslot, rslot):
        pltpu.make_async_copy(rows_v.at[rslot], agg_sh.at[dst_v.at[mslot]],
                              ssems[mslot]).wait()

    zb = rows_v.at[0]

    def zrow(i, carry):
        for cc in range(H // 16):
            zb[i, pl.ds(cc * 16, 16)] = jnp.zeros((16,), jnp.float32)
        return carry

    lax.fori_loop(0, EK, zrow, 0)

    def zcopy(i, carry):
        pltpu.sync_copy(zb, agg_sh.at[pl.ds(s * RZ + i * EK, EK)])
        return carry

    lax.fori_loop(0, RZ // EK, zcopy, 0)
    plsc.subcore_barrier()

    wch = jnp.where(c == 0, s * C0, NS * C0 + s * C1)
    nch = jnp.where(c == 0, C0, C1)
    kmax = nch // MRING

    for slot in range(5):
        meta_issue(slot, slot, wch)
    meta_wait(0, 0, wch)
    gather_issue(0, 0)
    meta_wait(1, 1, wch)
    gather_issue(1, 1)

    def body(k, carry):
        for b in range(MRING):
            t = k * MRING + b
            rb = b % NBUF
            gmb = (b + 2) % MRING
            grb = (b + 2) % NBUF
            wmb = (b + 6) % MRING
            pmb = (b + 5) % MRING

            def w_scat(wmb=wmb, wrb=grb):
                scatter_wait(wmb, wrb)

            def pf_gather(gmb=gmb, grb=grb, t=t):
                meta_wait(gmb, t + 2, wch)
                gather_issue(gmb, grb)

            def pf_meta(pmb=pmb, t=t):
                meta_issue(pmb, t + 5, wch)

            if b < 2:
                pl.when(k > 0)(w_scat)
            else:
                w_scat()
            if b < 6:
                pf_gather()
            else:
                pl.when(k < kmax - 1)(pf_gather)
            if b < 3:
                pf_meta()
            else:
                pl.when(k < kmax - 1)(pf_meta)
            gather_wait(b, rb)
            _scale_chunk(rows_v.at[rb], ew_v.at[b])
            scatter_issue(b, rb)
        return carry

    lax.fori_loop(0, kmax, body, 0)
    scatter_wait(6, 2)
    scatter_wait(7, 3)
    plsc.subcore_barrier()

    def wb(i, carry):
        pltpu.sync_copy(agg_sh.at[pl.ds(s * RZ + i * EK, EK)],
                        out_hbm.at[c, pl.ds(s * RZ + i * EK, EK)])
        return carry

    lax.fori_loop(0, RZ // EK, wb, 0)


_edge_pass = pl.kernel(
    _edge_body,
    out_type=jax.ShapeDtypeStruct((NC, NP, H), jnp.float32),
    mesh=_mesh,
    scratch_types=[
        pltpu.VMEM((MRING, EK), jnp.int32),
        pltpu.VMEM((MRING, EK), jnp.int32),
        pltpu.VMEM((MRING, EK), jnp.float32),
        pltpu.VMEM((NBUF, EK, H), jnp.float32),
        pltpu.VMEM_SHARED((NP, H), jnp.float32),
    ] + [pltpu.SemaphoreType.DMA] * (NBUF + 2 * MRING),
)


def _gru_body(a_ref, h_ref, bat_ref, nxt_ref, wggc_ref, wih_ref, whh_ref,
              bih_ref, bhh_ref, v_ref, sl_ref, cnt_ref):
    i = pl.program_id(0)
    agg = a_ref[0] + a_ref[1]
    h = h_ref[...]
    wc = lax.dot_general(wggc_ref[...], wih_ref[...], (((1,), (1,)), ((), ())))
    gi = lax.dot(agg, wc, preferred_element_type=jnp.float32) + bih_ref[...]
    gh = lax.dot_general(h, whh_ref[...], (((1,), (1,)), ((), ()))) + bhh_ref[...]
    r = jax.nn.sigmoid(gi[:, :H] + gh[:, :H])
    zg = jax.nn.sigmoid(gi[:, H:2 * H] + gh[:, H:2 * H])
    ng = jnp.tanh(gi[:, 2 * H:] + r * gh[:, 2 * H:])
    v = (1.0 - zg) * ng + zg * h
    v_ref[...] = v
    bat = bat_ref[0, 0, :]
    oh = (bat[:, None] == lax.broadcasted_iota(jnp.int32, (RB, B), 1)).astype(jnp.float32)
    islast = (bat != nxt_ref[0, 0, :]).astype(jnp.float32)

    @pl.when(i == 0)
    def _():
        sl_ref[...] = jnp.zeros_like(sl_ref)
        cnt_ref[...] = jnp.zeros_like(cnt_ref)

    sl_ref[...] += lax.dot_general(oh * islast[:, None], v, (((0,), (0,)), ((), ())))
    cnt_ref[...] += lax.dot_general(oh, jnp.ones((RB, 1), jnp.float32),
                                    (((0,), (0,)), ((), ())))


def _gru(agg2, h, bat2, nxt2, wggc, wih, whh, bih, bhh):
    return pl.pallas_call(
        _gru_body,
        grid=(NBLK,),
        in_specs=[
            pl.BlockSpec((NC, RB, H), lambda i: (0, i, 0)),
            pl.BlockSpec((RB, H), lambda i: (i, 0)),
            pl.BlockSpec((1, 1, RB), lambda i: (i, 0, 0)),
            pl.BlockSpec((1, 1, RB), lambda i: (i, 0, 0)),
            pl.BlockSpec((H, H), lambda i: (0, 0)),
            pl.BlockSpec((3 * H, H), lambda i: (0, 0)),
            pl.BlockSpec((3 * H, H), lambda i: (0, 0)),
            pl.BlockSpec((1, 3 * H), lambda i: (0, 0)),
            pl.BlockSpec((1, 3 * H), lambda i: (0, 0)),
        ],
        out_specs=[
            pl.BlockSpec((RB, H), lambda i: (i, 0)),
            pl.BlockSpec((B, H), lambda i: (0, 0)),
            pl.BlockSpec((B, 1), lambda i: (0, 0)),
        ],
        out_shape=[
            jax.ShapeDtypeStruct((NP, H), jnp.float32),
            jax.ShapeDtypeStruct((B, H), jnp.float32),
            jax.ShapeDtypeStruct((B, 1), jnp.float32),
        ],
    )(agg2, h, bat2, nxt2, wggc, wih, whh, bih, bhh)


def _att_body(v_ref, bat_ref, sl_ref, cnt_ref, vh_ref, w1_ref, b1_ref,
              w2_ref, b2_ref, qw_ref, qb_ref, w3_ref, b3_ref,
              sg_ref, sh_ref):
    i = pl.program_id(0)
    sl = sl_ref[...] + (cnt_ref[...] == 0.0).astype(jnp.float32) * vh_ref[0:1, :]
    bat = bat_ref[0, 0, :]
    oh = (bat[:, None] == lax.broadcasted_iota(jnp.int32, (RB, B), 1)).astype(jnp.float32)
    v = v_ref[...]
    vn = lax.dot(oh, sl, preferred_element_type=jnp.float32)
    pre = (lax.dot_general(vn, w1_ref[...], (((1,), (1,)), ((), ()))) + b1_ref[...]
           + lax.dot_general(v, w2_ref[...], (((1,), (1,)), ((), ()))) + b2_ref[...])
    sig = jax.nn.sigmoid(pre)
    alpha = jnp.sum(sig * qw_ref[...], axis=1, keepdims=True) + qb_ref[...]

    @pl.when(i == 0)
    def _():
        sg_ref[...] = jnp.zeros_like(sg_ref)

    sg_ref[...] += lax.dot_general(oh, alpha * v, (((0,), (0,)), ((), ())))

    @pl.when(i == NBLK - 1)
    def _():
        cat = jnp.concatenate([sl, sg_ref[...]], axis=1)
        sh_ref[...] = (lax.dot_general(cat, w3_ref[...], (((1,), (1,)), ((), ())))
                       + b3_ref[...])


def _att(v, bat2, sl, cnt, w1, b1, w2, b2, qw, qb, w3, b3):
    return pl.pallas_call(
        _att_body,
        grid=(NBLK,),
        in_specs=[
            pl.BlockSpec((RB, H), lambda i: (i, 0)),
            pl.BlockSpec((1, 1, RB), lambda i: (i, 0, 0)),
            pl.BlockSpec((B, H), lambda i: (0, 0)),
            pl.BlockSpec((B, 1), lambda i: (0, 0)),
            pl.BlockSpec((8, H), lambda i: (0, 0)),
            pl.BlockSpec((H, H), lambda i: (0, 0)),
            pl.BlockSpec((1, H), lambda i: (0, 0)),
            pl.BlockSpec((H, H), lambda i: (0, 0)),
            pl.BlockSpec((1, H), lambda i: (0, 0)),
            pl.BlockSpec((1, H), lambda i: (0, 0)),
            pl.BlockSpec((1, 1), lambda i: (0, 0)),
            pl.BlockSpec((H, 2 * H), lambda i: (0, 0)),
            pl.BlockSpec((1, H), lambda i: (0, 0)),
        ],
        out_specs=[
            pl.BlockSpec((B, H), lambda i: (0, 0)),
            pl.BlockSpec((B, H), lambda i: (0, 0)),
        ],
        out_shape=[
            jax.ShapeDtypeStruct((B, H), jnp.float32),
            jax.ShapeDtypeStruct((B, H), jnp.float32),
        ],
    )(v, bat2, sl, cnt, v, w1, b1, w2, b2, qw, qb, w3, b3)


def _z_body(sh_ref, emb_ref, z_ref):
    z_ref[...] = lax.dot_general(sh_ref[...], emb_ref[...], (((1,), (1,)), ((), ())))


def _zproj(sh, embed):
    return pl.pallas_call(
        _z_body,
        grid=(VG,),
        in_specs=[
            pl.BlockSpec((B, H), lambda i: (0, 0)),
            pl.BlockSpec((VB, H), lambda i: (i, 0)),
        ],
        out_specs=pl.BlockSpec((B, VB), lambda i: (0, i)),
        out_shape=jax.ShapeDtypeStruct((B, VOC), jnp.float32),
    )(sh, embed)


def kernel(x, edge_index, edge_weight, batch, embed, W_ggc, W_ih, W_hh, b_ih, b_hh,
           W1_w, W1_b, W2_w, W2_b, q_w, q_b, W3_w, W3_b):
    x_pad = jnp.concatenate([x.astype(jnp.int32),
                             jnp.arange(NP - N, dtype=jnp.int32)])
    h = _embed_gather(x_pad, embed)
    epad = jnp.arange(EP - E, dtype=jnp.int32) % N
    src = jnp.concatenate([edge_index[0].astype(jnp.int32), epad])
    dst = jnp.concatenate([edge_index[1].astype(jnp.int32), epad])
    ew2 = jnp.concatenate([edge_weight, jnp.zeros((EP - E,), jnp.float32)])
    agg2 = _edge_pass(h, src, dst, ew2)

    batch_pad = jnp.concatenate([batch.astype(jnp.int32),
                                 jnp.full((NP - N,), B, jnp.int32)])
    batch_nxt = jnp.concatenate([batch_pad[1:], jnp.full((1,), B, jnp.int32)])
    bat2 = batch_pad.reshape(NBLK, 1, RB)
    nxt2 = batch_nxt.reshape(NBLK, 1, RB)
    v, sl_raw, cnt = _gru(agg2, h, bat2, nxt2, W_ggc, W_ih, W_hh,
                          b_ih.reshape(1, 3 * H), b_hh.reshape(1, 3 * H))
    sg, sh = _att(v, bat2, sl_raw, cnt, W1_w, W1_b.reshape(1, H),
                  W2_w, W2_b.reshape(1, H), q_w.reshape(1, H),
                  q_b.reshape(1, 1), W3_w, W3_b.reshape(1, H))
    del sg
    return _zproj(sh, embed)
